# baseline (speedup 1.0000x reference)
import functools
import math

import jax
import jax.numpy as jnp
import numpy as np
from jax import lax
from jax.experimental import pallas as pl
from jax.experimental.pallas import tpu as pltpu
from jax.experimental.pallas import tpu_sc as plsc

_T23 = np.float32(math.tan(math.radians(23.0)))
_T68 = np.float32(math.tan(math.radians(68.0)))
_T113 = np.float32(math.tan(math.radians(113.0)))
_T158 = np.float32(math.tan(math.radians(158.0)))

_PAGE = 128
_PAGES_PER_CHUNK = 16
_C = _PAGE * _PAGES_PER_CHUNK


def _pick(dv, sl, zone, base=0):
    d = [dv[base + k, sl] for k in range(8)]
    m0 = (zone & 1) != 0
    m1 = (zone & 2) != 0
    m2 = (zone & 4) != 0
    t01 = jnp.where(m0, d[1], d[0])
    t23 = jnp.where(m0, d[3], d[2])
    t45 = jnp.where(m0, d[5], d[4])
    t67 = jnp.where(m0, d[7], d[6])
    t03 = jnp.where(m1, t23, t01)
    t47 = jnp.where(m1, t67, t45)
    return jnp.where(m2, t47, t03)


def _zone(x, y):
    ay = jnp.abs(y)
    s = (
        jnp.where(ay >= x * _T23, 1, 0)
        + jnp.where(ay >= x * _T68, 1, 0)
        + jnp.where(ay <= x * _T113, 1, 0)
        + jnp.where(ay <= x * _T158, 1, 0)
    )
    return (10 + jnp.where(y < 0, s, -s)) & 7


def _make_sc_call(B):
    info = plsc.get_sparse_core_info()
    NC, NS = info.num_cores, info.num_subcores
    NW = NC * NS
    P = B // _PAGE
    per = P // NW
    rem = P - NW * per
    n_chunks = -(-(per + 1) // _PAGES_PER_CHUNK)

    mesh = plsc.VectorSubcoreMesh(core_axis_name="c", subcore_axis_name="s")

    @functools.partial(
        pl.kernel,
        out_type=jax.ShapeDtypeStruct((B,), jnp.float32),
        mesh=mesh,
        scratch_types=[
            pltpu.VMEM((3, _C), jnp.float32),
            pltpu.VMEM((3, _C), jnp.float32),
            pltpu.VMEM((3, _C), jnp.float32),
            pltpu.VMEM((3, _C), jnp.float32),
            pltpu.VMEM((8, _C), jnp.float32),
            pltpu.VMEM((8, _C), jnp.float32),
            pltpu.VMEM((_C,), jnp.float32),
            pltpu.VMEM((_C,), jnp.float32),
            pltpu.VMEM((14, _PAGE), jnp.float32),
            pltpu.VMEM((_PAGE,), jnp.float32),
            pltpu.SemaphoreType.DMA,
            pltpu.SemaphoreType.DMA,
            pltpu.SemaphoreType.DMA,
            pltpu.SemaphoreType.DMA,
        ],
        compiler_params=pltpu.CompilerParams(needs_layout_passes=False),
    )
    def sc_call(z1t, z2t, dirt, tailtl, out_hbm,
                z1v0, z1v1, z2v0, z2v1, dirv0, dirv1, outv0, outv1,
                tailw, outw,
                sem_in0, sem_in1, sem_out0, sem_out1):
        cid = lax.axis_index("c")
        sid = lax.axis_index("s")
        wid = sid * NC + cid
        cnt = per + jnp.where(wid < rem, 1, 0)
        p0 = wid * per + jnp.minimum(wid, rem)

        z1v = (z1v0, z1v1)
        z2v = (z2v0, z2v1)
        dirv = (dirv0, dirv1)
        outv = (outv0, outv1)
        sems_in = (sem_in0, sem_in1)
        sems_out = (sem_out0, sem_out1)

        def run_chunk(z1b, z2b, db, ob, n_groups):
            @plsc.parallel_loop(0, n_groups, unroll=8)
            def _chunk_loop(j):
                sl = pl.ds(j * 16, 16)
                x = z2b[0, sl] - z1b[0, sl]
                y = z1b[2, sl] - z2b[2, sl]
                ob[sl] = _pick(db, sl, _zone(x, y))

        def chunk_slice(i):
            pstart = p0 + jnp.minimum(i * _PAGES_PER_CHUNK,
                                      cnt - _PAGES_PER_CHUNK)
            return pl.ds(pstart * _PAGE, _C)

        def start_in(i):
            b = i % 2
            sl = chunk_slice(i)
            return (
                pltpu.async_copy(z1t.at[:, sl], z1v[b], sems_in[b]),
                pltpu.async_copy(z2t.at[:, sl], z2v[b], sems_in[b]),
                pltpu.async_copy(dirt.at[:, sl], dirv[b], sems_in[b]),
            )

        pend_in = start_in(0)
        pend_out = [None, None]
        for i in range(n_chunks):
            b = i % 2
            nxt = start_in(i + 1) if i + 1 < n_chunks else None
            for d in pend_in:
                d.wait()
            pend_in = nxt
            if pend_out[b] is not None:
                pend_out[b].wait()
            run_chunk(z1v[b], z2v[b], dirv[b], outv[b], _C // 16)
            pend_out[b] = pltpu.async_copy(outv[b],
                                           out_hbm.at[chunk_slice(i)],
                                           sems_out[b])
        for d in pend_out:
            if d is not None:
                d.wait()

        @pl.when(wid == NW - 1)
        def _tail():
            pltpu.sync_copy(tailtl, tailw)

            @plsc.parallel_loop(0, _PAGE // 16, unroll=8)
            def _tail_loop(j):
                sl = pl.ds(j * 16, 16)
                x = tailw[3, sl] - tailw[0, sl]
                y = tailw[2, sl] - tailw[5, sl]
                outw[sl] = _pick(tailw, sl, _zone(x, y), base=6)

            pltpu.sync_copy(outw, out_hbm.at[pl.ds(B - _PAGE, _PAGE)])

    return sc_call


@jax.jit
def kernel(z_1, z_2, dir):
    B = dir.shape[0]
    sc_call = _make_sc_call(B)
    z1t, z2t, dirt = z_1.T, z_2.T, dir.T
    tail = jnp.concatenate(
        [z1t[:, B - _PAGE:], z2t[:, B - _PAGE:], dirt[:, B - _PAGE:]], axis=0)
    return sc_call(z1t, z2t, dirt, tail)


# speedup vs baseline: 1.0509x; 1.0509x over previous
import functools
import math

import jax
import jax.numpy as jnp
import numpy as np
from jax import lax
from jax.experimental import pallas as pl
from jax.experimental.pallas import tpu as pltpu
from jax.experimental.pallas import tpu_sc as plsc

_T23 = np.float32(math.tan(math.radians(23.0)))
_T68 = np.float32(math.tan(math.radians(68.0)))
_T113 = np.float32(math.tan(math.radians(113.0)))
_T158 = np.float32(math.tan(math.radians(158.0)))

_PAGE = 128
_PAGES_PER_CHUNK = 24
_C = _PAGE * _PAGES_PER_CHUNK


def _pick(dv, sl, zone, base=0):
    d = [dv[base + k, sl] for k in range(8)]
    m0 = (zone & 1) != 0
    m1 = (zone & 2) != 0
    m2 = (zone & 4) != 0
    t01 = jnp.where(m0, d[1], d[0])
    t23 = jnp.where(m0, d[3], d[2])
    t45 = jnp.where(m0, d[5], d[4])
    t67 = jnp.where(m0, d[7], d[6])
    t03 = jnp.where(m1, t23, t01)
    t47 = jnp.where(m1, t67, t45)
    return jnp.where(m2, t47, t03)


def _zone(x, y):
    ay = jnp.abs(y)
    s = (
        jnp.where(ay >= x * _T23, 1, 0)
        + jnp.where(ay >= x * _T68, 1, 0)
        + jnp.where(ay <= x * _T113, 1, 0)
        + jnp.where(ay <= x * _T158, 1, 0)
    )
    return (10 + jnp.where(y < 0, s, -s)) & 7


def _make_sc_call(B):
    info = plsc.get_sparse_core_info()
    NC, NS = info.num_cores, info.num_subcores
    NW = NC * NS
    P = B // _PAGE
    per = P // NW
    rem = P - NW * per
    n_chunks = -(-(per + 1) // _PAGES_PER_CHUNK)

    mesh = plsc.VectorSubcoreMesh(core_axis_name="c", subcore_axis_name="s")

    @functools.partial(
        pl.kernel,
        out_type=jax.ShapeDtypeStruct((B,), jnp.float32),
        mesh=mesh,
        scratch_types=[
            pltpu.VMEM((3, _C), jnp.float32),
            pltpu.VMEM((3, _C), jnp.float32),
            pltpu.VMEM((3, _C), jnp.float32),
            pltpu.VMEM((3, _C), jnp.float32),
            pltpu.VMEM((8, _C), jnp.float32),
            pltpu.VMEM((8, _C), jnp.float32),
            pltpu.VMEM((_C,), jnp.float32),
            pltpu.VMEM((_C,), jnp.float32),
            pltpu.VMEM((14, _PAGE), jnp.float32),
            pltpu.VMEM((_PAGE,), jnp.float32),
            pltpu.SemaphoreType.DMA,
            pltpu.SemaphoreType.DMA,
            pltpu.SemaphoreType.DMA,
            pltpu.SemaphoreType.DMA,
        ],
        compiler_params=pltpu.CompilerParams(needs_layout_passes=False),
    )
    def sc_call(z1t, z2t, dirt, tailtl, out_hbm,
                z1v0, z1v1, z2v0, z2v1, dirv0, dirv1, outv0, outv1,
                tailw, outw,
                sem_in0, sem_in1, sem_out0, sem_out1):
        cid = lax.axis_index("c")
        sid = lax.axis_index("s")
        wid = sid * NC + cid
        cnt = per + jnp.where(wid < rem, 1, 0)
        p0 = wid * per + jnp.minimum(wid, rem)

        z1v = (z1v0, z1v1)
        z2v = (z2v0, z2v1)
        dirv = (dirv0, dirv1)
        outv = (outv0, outv1)
        sems_in = (sem_in0, sem_in1)
        sems_out = (sem_out0, sem_out1)

        def run_chunk(z1b, z2b, db, ob, n_groups):
            @plsc.parallel_loop(0, n_groups, unroll=4)
            def _chunk_loop(j):
                sl = pl.ds(j * 16, 16)
                x = z2b[0, sl] - z1b[0, sl]
                y = z1b[2, sl] - z2b[2, sl]
                ob[sl] = _pick(db, sl, _zone(x, y))

        def chunk_slice(i):
            pstart = p0 + jnp.minimum(i * _PAGES_PER_CHUNK,
                                      cnt - _PAGES_PER_CHUNK)
            return pl.ds(pstart * _PAGE, _C)

        def start_in(i):
            b = i % 2
            sl = chunk_slice(i)
            return (
                pltpu.async_copy(z1t.at[:, sl], z1v[b], sems_in[b]),
                pltpu.async_copy(z2t.at[:, sl], z2v[b], sems_in[b]),
                pltpu.async_copy(dirt.at[:, sl], dirv[b], sems_in[b]),
            )

        pend_in = start_in(0)
        pend_out = [None, None]
        for i in range(n_chunks):
            b = i % 2
            nxt = start_in(i + 1) if i + 1 < n_chunks else None
            for d in pend_in:
                d.wait()
            pend_in = nxt
            if pend_out[b] is not None:
                pend_out[b].wait()
            run_chunk(z1v[b], z2v[b], dirv[b], outv[b], _C // 16)
            pend_out[b] = pltpu.async_copy(outv[b],
                                           out_hbm.at[chunk_slice(i)],
                                           sems_out[b])
        for d in pend_out:
            if d is not None:
                d.wait()

        @pl.when(wid == NW - 1)
        def _tail():
            pltpu.sync_copy(tailtl, tailw)

            @plsc.parallel_loop(0, _PAGE // 16, unroll=4)
            def _tail_loop(j):
                sl = pl.ds(j * 16, 16)
                x = tailw[3, sl] - tailw[0, sl]
                y = tailw[2, sl] - tailw[5, sl]
                outw[sl] = _pick(tailw, sl, _zone(x, y), base=6)

            pltpu.sync_copy(outw, out_hbm.at[pl.ds(B - _PAGE, _PAGE)])

    return sc_call


@jax.jit
def kernel(z_1, z_2, dir):
    B = dir.shape[0]
    sc_call = _make_sc_call(B)
    z1t, z2t, dirt = z_1.T, z_2.T, dir.T
    tail = jnp.concatenate(
        [z1t[:, B - _PAGE:], z2t[:, B - _PAGE:], dirt[:, B - _PAGE:]], axis=0)
    return sc_call(z1t, z2t, dirt, tail)


# speedup vs baseline: 1.0997x; 1.0464x over previous
import functools
import math

import jax
import jax.numpy as jnp
import numpy as np
from jax import lax
from jax.experimental import pallas as pl
from jax.experimental.pallas import tpu as pltpu
from jax.experimental.pallas import tpu_sc as plsc

_T23 = np.float32(math.tan(math.radians(23.0)))
_T68 = np.float32(math.tan(math.radians(68.0)))
_T113 = np.float32(math.tan(math.radians(113.0)))
_T158 = np.float32(math.tan(math.radians(158.0)))

_PAGE = 128
_PAGES_PER_CHUNK = 28
_C = _PAGE * _PAGES_PER_CHUNK


def _pick(dv, sl, zone, base=0):
    d = [dv[base + k, sl] for k in range(8)]
    m0 = (zone & 1) != 0
    m1 = (zone & 2) != 0
    m2 = (zone & 4) != 0
    t01 = jnp.where(m0, d[1], d[0])
    t23 = jnp.where(m0, d[3], d[2])
    t45 = jnp.where(m0, d[5], d[4])
    t67 = jnp.where(m0, d[7], d[6])
    t03 = jnp.where(m1, t23, t01)
    t47 = jnp.where(m1, t67, t45)
    return jnp.where(m2, t47, t03)


def _zone(x, y):
    ay = jnp.abs(y)
    s = (
        jnp.where(ay >= x * _T23, 1, 0)
        + jnp.where(ay >= x * _T68, 1, 0)
        + jnp.where(ay <= x * _T113, 1, 0)
        + jnp.where(ay <= x * _T158, 1, 0)
    )
    return (10 + jnp.where(y < 0, s, -s)) & 7


def _make_sc_call(B):
    info = plsc.get_sparse_core_info()
    NC, NS = info.num_cores, info.num_subcores
    NW = NC * NS
    P = B // _PAGE
    per = P // NW
    rem = P - NW * per
    n_chunks = -(-(per + 1) // _PAGES_PER_CHUNK)

    mesh = plsc.VectorSubcoreMesh(core_axis_name="c", subcore_axis_name="s")

    @functools.partial(
        pl.kernel,
        out_type=jax.ShapeDtypeStruct((B,), jnp.float32),
        mesh=mesh,
        scratch_types=[
            pltpu.VMEM((3, _C), jnp.float32),
            pltpu.VMEM((3, _C), jnp.float32),
            pltpu.VMEM((3, _C), jnp.float32),
            pltpu.VMEM((3, _C), jnp.float32),
            pltpu.VMEM((8, _C), jnp.float32),
            pltpu.VMEM((8, _C), jnp.float32),
            pltpu.VMEM((_C,), jnp.float32),
            pltpu.VMEM((_C,), jnp.float32),
            pltpu.VMEM((14, _PAGE), jnp.float32),
            pltpu.VMEM((_PAGE,), jnp.float32),
            pltpu.SemaphoreType.DMA,
            pltpu.SemaphoreType.DMA,
            pltpu.SemaphoreType.DMA,
            pltpu.SemaphoreType.DMA,
        ],
        compiler_params=pltpu.CompilerParams(needs_layout_passes=False),
    )
    def sc_call(z1t, z2t, dirt, tailtl, out_hbm,
                z1v0, z1v1, z2v0, z2v1, dirv0, dirv1, outv0, outv1,
                tailw, outw,
                sem_in0, sem_in1, sem_out0, sem_out1):
        cid = lax.axis_index("c")
        sid = lax.axis_index("s")
        wid = sid * NC + cid
        cnt = per + jnp.where(wid < rem, 1, 0)
        p0 = wid * per + jnp.minimum(wid, rem)

        z1v = (z1v0, z1v1)
        z2v = (z2v0, z2v1)
        dirv = (dirv0, dirv1)
        outv = (outv0, outv1)
        sems_in = (sem_in0, sem_in1)
        sems_out = (sem_out0, sem_out1)

        def run_chunk(z1b, z2b, db, ob, n_groups):
            @plsc.parallel_loop(0, n_groups, unroll=4)
            def _chunk_loop(j):
                sl = pl.ds(j * 16, 16)
                x = z2b[0, sl] - z1b[0, sl]
                y = z1b[2, sl] - z2b[2, sl]
                ob[sl] = _pick(db, sl, _zone(x, y))

        def chunk_slice(i):
            pstart = p0 + jnp.minimum(i * _PAGES_PER_CHUNK,
                                      cnt - _PAGES_PER_CHUNK)
            return pl.ds(pstart * _PAGE, _C)

        def start_in(i):
            b = i % 2
            sl = chunk_slice(i)
            return (
                pltpu.async_copy(z1t.at[:, sl], z1v[b], sems_in[b]),
                pltpu.async_copy(z2t.at[:, sl], z2v[b], sems_in[b]),
                pltpu.async_copy(dirt.at[:, sl], dirv[b], sems_in[b]),
            )

        pend_in = start_in(0)
        pend_out = [None, None]
        for i in range(n_chunks):
            b = i % 2
            nxt = start_in(i + 1) if i + 1 < n_chunks else None
            for d in pend_in:
                d.wait()
            pend_in = nxt
            if pend_out[b] is not None:
                pend_out[b].wait()
            run_chunk(z1v[b], z2v[b], dirv[b], outv[b], _C // 16)
            pend_out[b] = pltpu.async_copy(outv[b],
                                           out_hbm.at[chunk_slice(i)],
                                           sems_out[b])
        for d in pend_out:
            if d is not None:
                d.wait()

        @pl.when(wid == NW - 1)
        def _tail():
            pltpu.sync_copy(tailtl, tailw)

            @plsc.parallel_loop(0, _PAGE // 16, unroll=4)
            def _tail_loop(j):
                sl = pl.ds(j * 16, 16)
                x = tailw[3, sl] - tailw[0, sl]
                y = tailw[2, sl] - tailw[5, sl]
                outw[sl] = _pick(tailw, sl, _zone(x, y), base=6)

            pltpu.sync_copy(outw, out_hbm.at[pl.ds(B - _PAGE, _PAGE)])

    return sc_call


@jax.jit
def kernel(z_1, z_2, dir):
    B = dir.shape[0]
    sc_call = _make_sc_call(B)
    z1t, z2t, dirt = z_1.T, z_2.T, dir.T
    tail = jnp.concatenate(
        [z1t[:, B - _PAGE:], z2t[:, B - _PAGE:], dirt[:, B - _PAGE:]], axis=0)
    return sc_call(z1t, z2t, dirt, tail)


# speedup vs baseline: 1.1168x; 1.0155x over previous
import functools
import math

import jax
import jax.numpy as jnp
import numpy as np
from jax import lax
from jax.experimental import pallas as pl
from jax.experimental.pallas import tpu as pltpu
from jax.experimental.pallas import tpu_sc as plsc

_T23 = np.float32(math.tan(math.radians(23.0)))
_T68 = np.float32(math.tan(math.radians(68.0)))
_T113 = np.float32(math.tan(math.radians(113.0)))
_T158 = np.float32(math.tan(math.radians(158.0)))

_PAGE = 128
_PAGES_PER_CHUNK = 20
_C = _PAGE * _PAGES_PER_CHUNK
_NBUF = 3


def _pick(dv, sl, zone, base=0):
    d = [dv[base + k, sl] for k in range(8)]
    m0 = (zone & 1) != 0
    m1 = (zone & 2) != 0
    m2 = (zone & 4) != 0
    t01 = jnp.where(m0, d[1], d[0])
    t23 = jnp.where(m0, d[3], d[2])
    t45 = jnp.where(m0, d[5], d[4])
    t67 = jnp.where(m0, d[7], d[6])
    t03 = jnp.where(m1, t23, t01)
    t47 = jnp.where(m1, t67, t45)
    return jnp.where(m2, t47, t03)


def _zone(x, y):
    ay = jnp.abs(y)
    s = (
        jnp.where(ay >= x * _T23, 1, 0)
        + jnp.where(ay >= x * _T68, 1, 0)
        + jnp.where(ay <= x * _T113, 1, 0)
        + jnp.where(ay <= x * _T158, 1, 0)
    )
    return (10 + jnp.where(y < 0, s, -s)) & 7


def _make_sc_call(B):
    info = plsc.get_sparse_core_info()
    NC, NS = info.num_cores, info.num_subcores
    NW = NC * NS
    P = B // _PAGE
    per = P // NW
    rem = P - NW * per
    n_chunks = -(-(per + 1) // _PAGES_PER_CHUNK)

    mesh = plsc.VectorSubcoreMesh(core_axis_name="c", subcore_axis_name="s")

    @functools.partial(
        pl.kernel,
        out_type=jax.ShapeDtypeStruct((B,), jnp.float32),
        mesh=mesh,
        scratch_types=(
            [pltpu.VMEM((3, _C), jnp.float32)] * _NBUF
            + [pltpu.VMEM((3, _C), jnp.float32)] * _NBUF
            + [pltpu.VMEM((8, _C), jnp.float32)] * _NBUF
            + [pltpu.VMEM((_C,), jnp.float32)] * 2
            + [
                pltpu.VMEM((14, _PAGE), jnp.float32),
                pltpu.VMEM((_PAGE,), jnp.float32),
            ]
            + [pltpu.SemaphoreType.DMA] * (_NBUF + 2)
        ),
        compiler_params=pltpu.CompilerParams(needs_layout_passes=False),
    )
    def sc_call(z1t, z2t, dirt, tailtl, out_hbm, *bufs):
        z1v = bufs[0:_NBUF]
        z2v = bufs[_NBUF:2 * _NBUF]
        dirv = bufs[2 * _NBUF:3 * _NBUF]
        outv = bufs[3 * _NBUF:3 * _NBUF + 2]
        tailw = bufs[3 * _NBUF + 2]
        outw = bufs[3 * _NBUF + 3]
        sems_in = bufs[3 * _NBUF + 4:4 * _NBUF + 4]
        sems_out = bufs[4 * _NBUF + 4:4 * _NBUF + 6]

        cid = lax.axis_index("c")
        sid = lax.axis_index("s")
        wid = sid * NC + cid
        cnt = per + jnp.where(wid < rem, 1, 0)
        p0 = wid * per + jnp.minimum(wid, rem)

        def run_chunk(z1b, z2b, db, ob, n_groups):
            @plsc.parallel_loop(0, n_groups, unroll=4)
            def _chunk_loop(j):
                sl = pl.ds(j * 16, 16)
                x = z2b[0, sl] - z1b[0, sl]
                y = z1b[2, sl] - z2b[2, sl]
                ob[sl] = _pick(db, sl, _zone(x, y))

        def chunk_slice(i):
            pstart = p0 + jnp.minimum(i * _PAGES_PER_CHUNK,
                                      cnt - _PAGES_PER_CHUNK)
            return pl.ds(pstart * _PAGE, _C)

        def start_in(i):
            b = i % _NBUF
            sl = chunk_slice(i)
            return (
                pltpu.async_copy(z1t.at[:, sl], z1v[b], sems_in[b]),
                pltpu.async_copy(z2t.at[:, sl], z2v[b], sems_in[b]),
                pltpu.async_copy(dirt.at[:, sl], dirv[b], sems_in[b]),
            )

        pend_in = [start_in(i) for i in range(min(_NBUF - 1, n_chunks))]
        pend_out = [None, None]
        for i in range(n_chunks):
            b = i % _NBUF
            ob = i % 2
            if i + _NBUF - 1 < n_chunks:
                pend_in.append(start_in(i + _NBUF - 1))
            for d in pend_in.pop(0):
                d.wait()
            if pend_out[ob] is not None:
                pend_out[ob].wait()
            run_chunk(z1v[b], z2v[b], dirv[b], outv[ob], _C // 16)
            pend_out[ob] = pltpu.async_copy(outv[ob],
                                            out_hbm.at[chunk_slice(i)],
                                            sems_out[ob])
        for d in pend_out:
            if d is not None:
                d.wait()

        @pl.when(wid == NW - 1)
        def _tail():
            pltpu.sync_copy(tailtl, tailw)

            @plsc.parallel_loop(0, _PAGE // 16, unroll=4)
            def _tail_loop(j):
                sl = pl.ds(j * 16, 16)
                x = tailw[3, sl] - tailw[0, sl]
                y = tailw[2, sl] - tailw[5, sl]
                outw[sl] = _pick(tailw, sl, _zone(x, y), base=6)

            pltpu.sync_copy(outw, out_hbm.at[pl.ds(B - _PAGE, _PAGE)])

    return sc_call


@jax.jit
def kernel(z_1, z_2, dir):
    B = dir.shape[0]
    sc_call = _make_sc_call(B)
    z1t, z2t, dirt = z_1.T, z_2.T, dir.T
    tail = jnp.concatenate(
        [z1t[:, B - _PAGE:], z2t[:, B - _PAGE:], dirt[:, B - _PAGE:]], axis=0)
    return sc_call(z1t, z2t, dirt, tail)
